# baseline (speedup 1.0000x reference)
import functools

import jax
import jax.numpy as jnp
from jax import lax
from jax.experimental import pallas as pl
from jax.experimental.pallas import tpu as pltpu
from jax.experimental.pallas import tpu_sc as plsc

N = 10000
E = 320000
D = 128
HID = 128

NC = 2
NS = 16
L = 16
NW = NC * NS
EPW = E // NW
CH = 80
NCHUNK = EPW // CH
NP = 10240
RPS = NP // NS

BLK = 512
GRID = (N + BLK - 1) // BLK


def _pack_bf16_pair(hi_f32, lo_f32):
    hi = lax.bitcast_convert_type(hi_f32.astype(jnp.bfloat16),
                                  jnp.uint16).astype(jnp.uint32) << 16
    lo = lax.bitcast_convert_type(lo_f32.astype(jnp.bfloat16),
                                  jnp.uint16).astype(jnp.uint32)
    return lax.bitcast_convert_type(hi | lo, jnp.int32)


def _pre_body(dst_ref, a_ref, c_ref, wab_ref, wcb_ref,
              alab_ref, arab_ref, alcb_ref, arcb_ref,
              fsab_ref, fscb_ref, elrab_ref, elrcb_ref):
    hp = lax.Precision.HIGHEST
    fa = jnp.dot(a_ref[...], wab_ref[...], precision=hp)
    fc = jnp.dot(c_ref[...], wcb_ref[...], precision=hp)
    fdab = jnp.dot(dst_ref[...], wab_ref[...], precision=hp)
    fdcb = jnp.dot(dst_ref[...], wcb_ref[...], precision=hp)
    fsab_ref[...] = fa
    fscb_ref[...] = fc
    el_ab = jnp.sum(fa * alab_ref[...], axis=1, keepdims=True)
    er_ab = jnp.sum(fdab * arab_ref[...], axis=1, keepdims=True)
    el_cb = jnp.sum(fc * alcb_ref[...], axis=1, keepdims=True)
    er_cb = jnp.sum(fdcb * arcb_ref[...], axis=1, keepdims=True)
    elrab_ref[...] = _pack_bf16_pair(el_ab, er_ab)
    elrcb_ref[...] = _pack_bf16_pair(el_cb, er_cb)


def _pre(dst_feat, src_a, src_c, w_ab, w_cb, al_ab, ar_ab, al_cb, ar_cb):
    feat_spec = pl.BlockSpec((BLK, D), lambda i: (i, 0))
    full_spec = pl.BlockSpec((D, D), lambda i: (0, 0))
    attn_spec = pl.BlockSpec((1, D), lambda i: (0, 0))
    col_spec = pl.BlockSpec((BLK, 1), lambda i: (i, 0))
    return pl.pallas_call(
        _pre_body,
        grid=(GRID,),
        in_specs=[feat_spec, feat_spec, feat_spec, full_spec, full_spec,
                  attn_spec, attn_spec, attn_spec, attn_spec],
        out_specs=[feat_spec, feat_spec, col_spec, col_spec],
        out_shape=[jax.ShapeDtypeStruct((N, D), jnp.float32),
                   jax.ShapeDtypeStruct((N, D), jnp.float32),
                   jax.ShapeDtypeStruct((N, 1), jnp.int32),
                   jax.ShapeDtypeStruct((N, 1), jnp.int32)],
    )(dst_feat, src_a, src_c, w_ab, w_cb, al_ab, ar_ab, al_cb, ar_cb)


def _sc_body(fsab_hbm, fscb_hbm, elrab_hbm, elrcb_hbm,
             sab_hbm, dab_hbm, scb_hbm, dcb_hbm, z128_hbm,
             pout_hbm, pden_hbm,
             elr_v, den_v, sidx_v, didx_v, rows_v,
             out_sh, gsem, ssem):
    cid = lax.axis_index("c")
    sid = lax.axis_index("s")
    wid = cid * NS + sid
    himask = jnp.full((L,), 0xFFFF0000, dtype=jnp.uint32).astype(jnp.int32)
    zvec = jnp.zeros((L,), jnp.float32)

    rels = [(0, fsab_hbm, elrab_hbm, sab_hbm, dab_hbm),
            (1, fscb_hbm, elrcb_hbm, scb_hbm, dcb_hbm)]
    for rel, fs_hbm, elr_hbm, s_hbm, d_hbm in rels:
        pltpu.sync_copy(z128_hbm.at[pl.ds(sid * RPS, RPS)],
                        out_sh.at[pl.ds(sid * RPS, RPS)])
        pltpu.sync_copy(elr_hbm, elr_v)

        @pl.loop(0, N, step=L)
        def _zero(i):
            den_v[pl.ds(i, L)] = zvec

        plsc.subcore_barrier()

        base = wid * EPW

        @pl.loop(0, NCHUNK)
        def _chunk(cix):
            off = base + cix * CH
            pltpu.sync_copy(s_hbm.at[pl.ds(off, CH)], sidx_v)
            pltpu.sync_copy(d_hbm.at[pl.ds(off, CH)], didx_v)
            gat = pltpu.async_copy(fs_hbm.at[sidx_v], rows_v, gsem)
            ws = []
            for g in range(CH // L):
                i_s = sidx_v[pl.ds(g * L, L)]
                i_d = didx_v[pl.ds(g * L, L)]
                v_s = plsc.load_gather(elr_v, [i_s])
                v_d = plsc.load_gather(elr_v, [i_d])
                el_s = plsc.bitcast(v_s & himask, jnp.float32)
                er_d = plsc.bitcast(v_d << 16, jnp.float32)
                x = el_s + er_d
                e = jnp.where(x >= 0, x, 0.2 * x)
                w = jnp.exp(e)
                plsc.addupdate_scatter(den_v, [i_d], w)
                ws.append(w)
            gat.wait()
            for g in range(CH // L):
                w = ws[g]
                for r in range(L):
                    row = g * L + r
                    wb = jnp.full((L,), w[r], dtype=jnp.float32)
                    for q in range(D // L):
                        rows_v[row, pl.ds(q * L, L)] = (
                            rows_v[row, pl.ds(q * L, L)] * wb)
            pltpu.async_copy(rows_v, out_sh.at[didx_v], ssem,
                             add=True).wait()

        plsc.subcore_barrier()
        pltpu.sync_copy(out_sh.at[pl.ds(sid * RPS, RPS)],
                        pout_hbm.at[rel, cid, pl.ds(sid * RPS, RPS)])
        pltpu.sync_copy(den_v, pden_hbm.at[rel, wid])
        plsc.subcore_barrier()


def _sc_edges(fs_ab, fs_cb, elr_ab, elr_cb,
              s_ab, d_ab, s_cb, d_cb):
    z128 = jnp.zeros((NP, D), jnp.float32)
    mesh = plsc.VectorSubcoreMesh(core_axis_name="c", subcore_axis_name="s")
    run = pl.kernel(
        _sc_body,
        compiler_params=pltpu.CompilerParams(needs_layout_passes=False),
        out_type=(jax.ShapeDtypeStruct((2, NC, NP, D), jnp.float32),
                  jax.ShapeDtypeStruct((2, NW, N), jnp.float32)),
        mesh=mesh,
        scratch_types=[
            pltpu.VMEM((N,), jnp.int32),
            pltpu.VMEM((N,), jnp.float32),
            pltpu.VMEM((CH,), jnp.int32),
            pltpu.VMEM((CH,), jnp.int32),
            pltpu.VMEM((CH, D), jnp.float32),
            pltpu.VMEM_SHARED((NP, D), jnp.float32),
            pltpu.SemaphoreType.DMA,
            pltpu.SemaphoreType.DMA,
        ],
    )
    return run(fs_ab, fs_cb, elr_ab, elr_cb,
               s_ab, d_ab, s_cb, d_cb, z128)


def _post1_body(pa0, pa1, pc0, pc1, da_ref, dc_ref,
                bab_ref, bcb_ref, w1_ref, b1_ref, w2_ref,
                zab_ref, zcb_ref, ssum_ref):
    i = pl.program_id(0)
    hp = lax.Precision.HIGHEST

    def mkz(p0, p1, d_ref, bias_ref):
        den = jnp.sum(d_ref[...], axis=0)[:, None]
        out = (p0[...] + p1[...]) / jnp.maximum(den, 1e-9)
        out = out + bias_ref[...]
        return jnp.where(out > 0, out, jnp.exp(jnp.minimum(out, 0.0)) - 1.0)

    za = mkz(pa0, pa1, da_ref, bab_ref)
    zc = mkz(pc0, pc1, dc_ref, bcb_ref)
    zab_ref[...] = za
    zcb_ref[...] = zc

    def score(z):
        h = jnp.tanh(jnp.dot(z, w1_ref[...], precision=hp) + b1_ref[...])
        return jnp.dot(h, w2_ref[...], precision=hp)

    rowids = i * BLK + lax.broadcasted_iota(jnp.int32, (BLK, 1), 0)
    mask = rowids < N
    sa = jnp.sum(jnp.where(mask, score(za), 0.0))
    sc = jnp.sum(jnp.where(mask, score(zc), 0.0))

    @pl.when(i == 0)
    def _():
        ssum_ref[...] = jnp.zeros_like(ssum_ref)

    ssum_ref[...] += jnp.concatenate(
        [sa.reshape(1, 1), sc.reshape(1, 1)], axis=1)


def _post1(pout, pden, bias_ab, bias_cb, w1, b1, w2):
    feat_spec = pl.BlockSpec((BLK, D), lambda i: (i, 0))
    den_spec = pl.BlockSpec((NW, BLK), lambda i: (0, i))
    bias_spec = pl.BlockSpec((1, D), lambda i: (0, 0))
    w1_spec = pl.BlockSpec((D, D), lambda i: (0, 0))
    w2_spec = pl.BlockSpec((D, 1), lambda i: (0, 0))
    ssum_spec = pl.BlockSpec((1, 2), lambda i: (0, 0))
    return pl.pallas_call(
        _post1_body,
        grid=(GRID,),
        in_specs=[feat_spec, feat_spec, feat_spec, feat_spec,
                  den_spec, den_spec,
                  bias_spec, bias_spec, w1_spec, bias_spec, w2_spec],
        out_specs=[feat_spec, feat_spec, ssum_spec],
        out_shape=[jax.ShapeDtypeStruct((N, D), jnp.float32),
                   jax.ShapeDtypeStruct((N, D), jnp.float32),
                   jax.ShapeDtypeStruct((1, 2), jnp.float32)],
    )(pout[0, 0], pout[0, 1], pout[1, 0], pout[1, 1],
      pden[0], pden[1],
      bias_ab, bias_cb, w1, b1, w2)


def _post2_body(zab_ref, zcb_ref, ssum_ref, z_ref, att_ref):
    s = ssum_ref[...] / N
    m = jnp.max(s)
    ex = jnp.exp(s - m)
    aw = ex / jnp.sum(ex)
    a0 = aw[0:1, 0:1]
    a1 = aw[0:1, 1:2]
    z_ref[...] = zab_ref[...] * a0 + zcb_ref[...] * a1
    att_ref[...] = aw


def _post2(zab, zcb, ssum):
    feat_spec = pl.BlockSpec((BLK, D), lambda i: (i, 0))
    ssum_spec = pl.BlockSpec((1, 2), lambda i: (0, 0))
    return pl.pallas_call(
        _post2_body,
        grid=(GRID,),
        in_specs=[feat_spec, feat_spec, ssum_spec],
        out_specs=[feat_spec, ssum_spec],
        out_shape=[jax.ShapeDtypeStruct((N, D), jnp.float32),
                   jax.ShapeDtypeStruct((1, 2), jnp.float32)],
    )(zab, zcb, ssum)


def kernel(dst_feat, src_feat_a, src_feat_c, edge_index_ab, edge_index_cb,
           W_ab, attn_l_ab, attn_r_ab, bias_ab,
           W_cb, attn_l_cb, attn_r_cb, bias_cb,
           W1_sem, b1_sem, W2_sem):
    al_ab = attn_l_ab.reshape(1, D)
    ar_ab = attn_r_ab.reshape(1, D)
    al_cb = attn_l_cb.reshape(1, D)
    ar_cb = attn_r_cb.reshape(1, D)

    fs_ab, fs_cb, elr_ab, elr_cb = _pre(
        dst_feat, src_feat_a, src_feat_c, W_ab, W_cb,
        al_ab, ar_ab, al_cb, ar_cb)

    pout, pden = _sc_edges(
        fs_ab, fs_cb,
        elr_ab.reshape(N), elr_cb.reshape(N),
        edge_index_ab[0], edge_index_ab[1],
        edge_index_cb[0], edge_index_cb[1])

    zab, zcb, ssum = _post1(pout, pden,
                            bias_ab.reshape(1, D), bias_cb.reshape(1, D),
                            W1_sem, b1_sem.reshape(1, D), W2_sem)

    z, att = _post2(zab, zcb, ssum)
    return (z, att.reshape(2))


# speedup vs baseline: 1.2756x; 1.2756x over previous
import functools

import jax
import jax.numpy as jnp
from jax import lax
from jax.experimental import pallas as pl
from jax.experimental.pallas import tpu as pltpu
from jax.experimental.pallas import tpu_sc as plsc

N = 10000
E = 320000
D = 128
HID = 128

NC = 2
NS = 16
L = 16
NW = NC * NS
EPW = E // NW
CH = 80
NCHUNK = EPW // CH
NP = 10240
RPS = NP // NS

BLK = 512
GRID = (N + BLK - 1) // BLK


def _pack_bf16_pair(hi_f32, lo_f32):
    hi = lax.bitcast_convert_type(hi_f32.astype(jnp.bfloat16),
                                  jnp.uint16).astype(jnp.uint32) << 16
    lo = lax.bitcast_convert_type(lo_f32.astype(jnp.bfloat16),
                                  jnp.uint16).astype(jnp.uint32)
    return lax.bitcast_convert_type(hi | lo, jnp.int32)


def _pre_body(dst_ref, a_ref, c_ref, wab_ref, wcb_ref,
              alab_ref, arab_ref, alcb_ref, arcb_ref,
              fsab_ref, fscb_ref, elrab_ref, elrcb_ref):
    hp = lax.Precision.HIGHEST
    fa = jnp.dot(a_ref[...], wab_ref[...], precision=hp)
    fc = jnp.dot(c_ref[...], wcb_ref[...], precision=hp)
    fdab = jnp.dot(dst_ref[...], wab_ref[...], precision=hp)
    fdcb = jnp.dot(dst_ref[...], wcb_ref[...], precision=hp)
    fsab_ref[...] = fa
    fscb_ref[...] = fc
    el_ab = jnp.sum(fa * alab_ref[...], axis=1, keepdims=True)
    er_ab = jnp.sum(fdab * arab_ref[...], axis=1, keepdims=True)
    el_cb = jnp.sum(fc * alcb_ref[...], axis=1, keepdims=True)
    er_cb = jnp.sum(fdcb * arcb_ref[...], axis=1, keepdims=True)
    elrab_ref[...] = _pack_bf16_pair(el_ab, er_ab)
    elrcb_ref[...] = _pack_bf16_pair(el_cb, er_cb)


def _pre(dst_feat, src_a, src_c, w_ab, w_cb, al_ab, ar_ab, al_cb, ar_cb):
    feat_spec = pl.BlockSpec((BLK, D), lambda i: (i, 0))
    full_spec = pl.BlockSpec((D, D), lambda i: (0, 0))
    attn_spec = pl.BlockSpec((1, D), lambda i: (0, 0))
    col_spec = pl.BlockSpec((BLK, 1), lambda i: (i, 0))
    return pl.pallas_call(
        _pre_body,
        grid=(GRID,),
        in_specs=[feat_spec, feat_spec, feat_spec, full_spec, full_spec,
                  attn_spec, attn_spec, attn_spec, attn_spec],
        out_specs=[feat_spec, feat_spec, col_spec, col_spec],
        out_shape=[jax.ShapeDtypeStruct((N, D), jnp.float32),
                   jax.ShapeDtypeStruct((N, D), jnp.float32),
                   jax.ShapeDtypeStruct((N, 1), jnp.int32),
                   jax.ShapeDtypeStruct((N, 1), jnp.int32)],
    )(dst_feat, src_a, src_c, w_ab, w_cb, al_ab, ar_ab, al_cb, ar_cb)


def _sc_body(fsab_hbm, fscb_hbm, elrab_hbm, elrcb_hbm,
             sab_hbm, dab_hbm, scb_hbm, dcb_hbm, z128_hbm,
             pout_hbm, pden_hbm,
             elr_v, den_v, sidxa_v, didxa_v, sidxb_v, didxb_v,
             rowsa_v, rowsb_v,
             out_sh, gsema, gsemb, ssema, ssemb):
    cid = lax.axis_index("c")
    sid = lax.axis_index("s")
    wid = cid * NS + sid
    himask = jnp.full((L,), 0xFFFF0000, dtype=jnp.uint32).astype(jnp.int32)
    zvec = jnp.zeros((L,), jnp.float32)

    rels = [(0, fsab_hbm, elrab_hbm, sab_hbm, dab_hbm),
            (1, fscb_hbm, elrcb_hbm, scb_hbm, dcb_hbm)]
    for rel, fs_hbm, elr_hbm, s_hbm, d_hbm in rels:
        pltpu.sync_copy(z128_hbm.at[pl.ds(sid * RPS, RPS)],
                        out_sh.at[pl.ds(sid * RPS, RPS)])
        pltpu.sync_copy(elr_hbm, elr_v)

        @pl.loop(0, N, step=L)
        def _zero(i):
            den_v[pl.ds(i, L)] = zvec

        plsc.subcore_barrier()

        base = wid * EPW

        def load_idx(cix, sidx_v, didx_v):
            off = base + cix * CH
            pltpu.sync_copy(s_hbm.at[pl.ds(off, CH)], sidx_v)
            pltpu.sync_copy(d_hbm.at[pl.ds(off, CH)], didx_v)

        def fire_gathers(sidx_v, rows_v, sem):
            for g in range(CH // L):
                i_s = sidx_v[pl.ds(g * L, L)]
                pltpu.async_copy(fs_hbm.at[i_s],
                                 rows_v.at[pl.ds(g * L, L)], sem)

        def drain(rows_v, sem):
            pltpu.make_async_copy(z128_hbm.at[pl.ds(0, CH)],
                                  rows_v, sem).wait()

        def process(sidx_v, didx_v, rows_v, ssem):
            for g in range(CH // L):
                i_s = sidx_v[pl.ds(g * L, L)]
                i_d = didx_v[pl.ds(g * L, L)]
                v_s = plsc.load_gather(elr_v, [i_s])
                v_d = plsc.load_gather(elr_v, [i_d])
                el_s = plsc.bitcast(v_s & himask, jnp.float32)
                er_d = plsc.bitcast(v_d << 16, jnp.float32)
                x = el_s + er_d
                e = jnp.where(x >= 0, x, 0.2 * x)
                w = jnp.exp(e)
                plsc.addupdate_scatter(den_v, [i_d], w)
                for r in range(L):
                    row = g * L + r
                    wb = jnp.full((L,), w[r], dtype=jnp.float32)
                    for q in range(D // L):
                        rows_v[row, pl.ds(q * L, L)] = (
                            rows_v[row, pl.ds(q * L, L)] * wb)
                pltpu.async_copy(rows_v.at[pl.ds(g * L, L)],
                                 out_sh.at[i_d], ssem, add=True)

        load_idx(0, sidxa_v, didxa_v)
        fire_gathers(sidxa_v, rowsa_v, gsema)

        @pl.loop(0, NCHUNK - 1, step=2)
        def _chunk(cix):
            load_idx(cix + 1, sidxb_v, didxb_v)

            @pl.when(cix > 0)
            def _():
                drain(rowsb_v, ssemb)
            fire_gathers(sidxb_v, rowsb_v, gsemb)

            drain(rowsa_v, gsema)
            process(sidxa_v, didxa_v, rowsa_v, ssema)

            load_idx(cix + 2, sidxa_v, didxa_v)
            drain(rowsb_v, gsemb)
            process(sidxb_v, didxb_v, rowsb_v, ssemb)

            drain(rowsa_v, ssema)
            fire_gathers(sidxa_v, rowsa_v, gsema)

        drain(rowsa_v, gsema)
        process(sidxa_v, didxa_v, rowsa_v, ssema)
        drain(rowsb_v, ssemb)
        drain(rowsa_v, ssema)

        plsc.subcore_barrier()
        pltpu.sync_copy(out_sh.at[pl.ds(sid * RPS, RPS)],
                        pout_hbm.at[rel, cid, pl.ds(sid * RPS, RPS)])
        pltpu.sync_copy(den_v, pden_hbm.at[rel, wid])
        plsc.subcore_barrier()


def _sc_edges(fs_ab, fs_cb, elr_ab, elr_cb,
              s_ab, d_ab, s_cb, d_cb):
    z128 = jnp.zeros((NP, D), jnp.float32)
    mesh = plsc.VectorSubcoreMesh(core_axis_name="c", subcore_axis_name="s")
    run = pl.kernel(
        _sc_body,
        compiler_params=pltpu.CompilerParams(needs_layout_passes=False),
        out_type=(jax.ShapeDtypeStruct((2, NC, NP, D), jnp.float32),
                  jax.ShapeDtypeStruct((2, NW, N), jnp.float32)),
        mesh=mesh,
        scratch_types=[
            pltpu.VMEM((N,), jnp.int32),
            pltpu.VMEM((N,), jnp.float32),
            pltpu.VMEM((CH,), jnp.int32),
            pltpu.VMEM((CH,), jnp.int32),
            pltpu.VMEM((CH,), jnp.int32),
            pltpu.VMEM((CH,), jnp.int32),
            pltpu.VMEM((CH, D), jnp.float32),
            pltpu.VMEM((CH, D), jnp.float32),
            pltpu.VMEM_SHARED((NP, D), jnp.float32),
            pltpu.SemaphoreType.DMA,
            pltpu.SemaphoreType.DMA,
            pltpu.SemaphoreType.DMA,
            pltpu.SemaphoreType.DMA,
        ],
    )
    return run(fs_ab, fs_cb, elr_ab, elr_cb,
               s_ab, d_ab, s_cb, d_cb, z128)


def _post1_body(pa0, pa1, pc0, pc1, da_ref, dc_ref,
                bab_ref, bcb_ref, w1_ref, b1_ref, w2_ref,
                zab_ref, zcb_ref, ssum_ref):
    i = pl.program_id(0)
    hp = lax.Precision.HIGHEST

    def mkz(p0, p1, d_ref, bias_ref):
        den = jnp.sum(d_ref[...], axis=0)[:, None]
        out = (p0[...] + p1[...]) / jnp.maximum(den, 1e-9)
        out = out + bias_ref[...]
        return jnp.where(out > 0, out, jnp.exp(jnp.minimum(out, 0.0)) - 1.0)

    za = mkz(pa0, pa1, da_ref, bab_ref)
    zc = mkz(pc0, pc1, dc_ref, bcb_ref)
    zab_ref[...] = za
    zcb_ref[...] = zc

    def score(z):
        h = jnp.tanh(jnp.dot(z, w1_ref[...], precision=hp) + b1_ref[...])
        return jnp.dot(h, w2_ref[...], precision=hp)

    rowids = i * BLK + lax.broadcasted_iota(jnp.int32, (BLK, 1), 0)
    mask = rowids < N
    sa = jnp.sum(jnp.where(mask, score(za), 0.0))
    sc = jnp.sum(jnp.where(mask, score(zc), 0.0))

    @pl.when(i == 0)
    def _():
        ssum_ref[...] = jnp.zeros_like(ssum_ref)

    ssum_ref[...] += jnp.concatenate(
        [sa.reshape(1, 1), sc.reshape(1, 1)], axis=1)


def _post1(pout, pden, bias_ab, bias_cb, w1, b1, w2):
    feat_spec = pl.BlockSpec((BLK, D), lambda i: (i, 0))
    den_spec = pl.BlockSpec((NW, BLK), lambda i: (0, i))
    bias_spec = pl.BlockSpec((1, D), lambda i: (0, 0))
    w1_spec = pl.BlockSpec((D, D), lambda i: (0, 0))
    w2_spec = pl.BlockSpec((D, 1), lambda i: (0, 0))
    ssum_spec = pl.BlockSpec((1, 2), lambda i: (0, 0))
    return pl.pallas_call(
        _post1_body,
        grid=(GRID,),
        in_specs=[feat_spec, feat_spec, feat_spec, feat_spec,
                  den_spec, den_spec,
                  bias_spec, bias_spec, w1_spec, bias_spec, w2_spec],
        out_specs=[feat_spec, feat_spec, ssum_spec],
        out_shape=[jax.ShapeDtypeStruct((N, D), jnp.float32),
                   jax.ShapeDtypeStruct((N, D), jnp.float32),
                   jax.ShapeDtypeStruct((1, 2), jnp.float32)],
    )(pout[0, 0], pout[0, 1], pout[1, 0], pout[1, 1],
      pden[0], pden[1],
      bias_ab, bias_cb, w1, b1, w2)


def _post2_body(zab_ref, zcb_ref, ssum_ref, z_ref, att_ref):
    s = ssum_ref[...] / N
    m = jnp.max(s)
    ex = jnp.exp(s - m)
    aw = ex / jnp.sum(ex)
    a0 = aw[0:1, 0:1]
    a1 = aw[0:1, 1:2]
    z_ref[...] = zab_ref[...] * a0 + zcb_ref[...] * a1
    att_ref[...] = aw


def _post2(zab, zcb, ssum):
    feat_spec = pl.BlockSpec((BLK, D), lambda i: (i, 0))
    ssum_spec = pl.BlockSpec((1, 2), lambda i: (0, 0))
    return pl.pallas_call(
        _post2_body,
        grid=(GRID,),
        in_specs=[feat_spec, feat_spec, ssum_spec],
        out_specs=[feat_spec, ssum_spec],
        out_shape=[jax.ShapeDtypeStruct((N, D), jnp.float32),
                   jax.ShapeDtypeStruct((1, 2), jnp.float32)],
    )(zab, zcb, ssum)


def kernel(dst_feat, src_feat_a, src_feat_c, edge_index_ab, edge_index_cb,
           W_ab, attn_l_ab, attn_r_ab, bias_ab,
           W_cb, attn_l_cb, attn_r_cb, bias_cb,
           W1_sem, b1_sem, W2_sem):
    al_ab = attn_l_ab.reshape(1, D)
    ar_ab = attn_r_ab.reshape(1, D)
    al_cb = attn_l_cb.reshape(1, D)
    ar_cb = attn_r_cb.reshape(1, D)

    fs_ab, fs_cb, elr_ab, elr_cb = _pre(
        dst_feat, src_feat_a, src_feat_c, W_ab, W_cb,
        al_ab, ar_ab, al_cb, ar_cb)

    pout, pden = _sc_edges(
        fs_ab, fs_cb,
        elr_ab.reshape(N), elr_cb.reshape(N),
        edge_index_ab[0], edge_index_ab[1],
        edge_index_cb[0], edge_index_cb[1])

    zab, zcb, ssum = _post1(pout, pden,
                            bias_ab.reshape(1, D), bias_cb.reshape(1, D),
                            W1_sem, b1_sem.reshape(1, D), W2_sem)

    z, att = _post2(zab, zcb, ssum)
    return (z, att.reshape(2))
